# baseline (speedup 1.0000x reference)
import jax
import jax.numpy as jnp
from jax import lax
from jax.experimental import pallas as pl
from jax.experimental.pallas import tpu as pltpu
from jax.experimental.pallas import tpu_sc as plsc

VOCAB = 1000000
MAXLEN = 200
EMBED = 64
BATCH = 1024
HALFW = EMBED // 2

NC, NS = 2, 16
NW = NC * NS
ROWS = BATCH * MAXLEN
RPW = ROWS // NW
CHUNK = 100
CPW = RPW // CHUNK
LANES = 16


def _body(x_ref, tok_ref, pos_ref, out_ref, pos_v, idx_v, pr_v, g_v, o_v,
          g_sem, o_sem):
    wid = lax.axis_index("s") * NC + lax.axis_index("c")
    base_chunk = wid * CPW
    lane = lax.iota(jnp.int32, LANES)

    pltpu.sync_copy(pos_ref, pos_v)
    pltpu.sync_copy(x_ref.at[pl.ds(base_chunk, CPW)], idx_v)

    def pr_body(c, carry):
        for h in (0, 1):
            for k in (0, 16, 32, 34):
                src = idx_v[c, pl.ds(h * 50 + k, LANES)]
                even = src * 2
                dst = 2 * k + lane * 2
                ref = pr_v.at[2 * c + h]
                plsc.store_scatter(ref, [dst], even)
                plsc.store_scatter(ref, [dst + 1], even + 1)
        return carry

    lax.fori_loop(0, CPW, pr_body, 0)

    def start_gathers(c, buf):
        pltpu.async_copy(
            tok_ref.at[pr_v.at[2 * c]],
            g_v.at[buf].at[pl.ds(0, CHUNK)], g_sem.at[buf])
        pltpu.async_copy(
            tok_ref.at[pr_v.at[2 * c + 1]],
            g_v.at[buf].at[pl.ds(CHUNK, CHUNK)], g_sem.at[buf])

    start_gathers(0, 0)
    start_gathers(1, 1)

    def chunk_body(c, carry):
        b = c % 2
        pltpu.make_async_copy(
            tok_ref.at[pr_v.at[2 * c]],
            g_v.at[b].at[pl.ds(0, CHUNK)], g_sem.at[b]).wait()
        pltpu.make_async_copy(
            tok_ref.at[pr_v.at[2 * c + 1]],
            g_v.at[b].at[pl.ds(CHUNK, CHUNK)], g_sem.at[b]).wait()

        @pl.when(c >= 2)
        def _():
            pltpu.make_async_copy(
                o_v.at[b], out_ref.at[0], o_sem.at[b]).wait()

        half = (c % 2) * CHUNK

        def row_body(r, carry2):
            p = half + r
            for j in range(2):
                g_row = 2 * r + j
                for d in range(HALFW // LANES):
                    o_v[b, r, pl.ds(j * HALFW + d * LANES, LANES)] = (
                        g_v[b, g_row, pl.ds(d * LANES, LANES)]
                        + pos_v[p, pl.ds(j * HALFW + d * LANES, LANES)])
            return carry2

        lax.fori_loop(0, CHUNK, row_body, 0, unroll=2)

        pltpu.async_copy(o_v.at[b], out_ref.at[base_chunk + c], o_sem.at[b])

        @pl.when(c + 2 < CPW)
        def _():
            start_gathers(c + 2, b)

        return carry

    lax.fori_loop(0, CPW, chunk_body, 0)

    pltpu.make_async_copy(o_v.at[0], out_ref.at[0], o_sem.at[0]).wait()
    pltpu.make_async_copy(o_v.at[1], out_ref.at[0], o_sem.at[1]).wait()


def kernel(x, token_table, pos_table):
    x2 = x.reshape(NW * CPW, CHUNK)
    table2 = token_table.reshape(2 * VOCAB, HALFW)
    mesh = plsc.VectorSubcoreMesh(
        core_axis_name="c", subcore_axis_name="s",
        num_cores=NC, num_subcores=NS)
    out = pl.kernel(
        _body,
        out_type=jax.ShapeDtypeStruct((NW * CPW, CHUNK, EMBED), jnp.float32),
        mesh=mesh,
        scratch_types=[
            pltpu.VMEM((MAXLEN, EMBED), jnp.float32),
            pltpu.VMEM((CPW, CHUNK), jnp.int32),
            pltpu.VMEM((2 * CPW, CHUNK), jnp.int32),
            pltpu.VMEM((2, 2 * CHUNK, HALFW), jnp.float32),
            pltpu.VMEM((2, CHUNK, EMBED), jnp.float32),
            pltpu.SemaphoreType.DMA((2,)),
            pltpu.SemaphoreType.DMA((2,)),
        ],
        compiler_params=pltpu.CompilerParams(
            use_tc_tiling_on_sc=False, needs_layout_passes=False),
    )(x2, table2, pos_table)
    return out.reshape(BATCH, MAXLEN, EMBED)
